# Initial kernel scaffold; baseline (speedup 1.0000x reference)
import jax
import jax.numpy as jnp
from jax.experimental import pallas as pl


def kernel(inputs, unnormalized_widths, unnormalized_heights, unnormalized_derivatives):
    raise NotImplementedError("write your pallas kernel here")


# speedup vs baseline: 24.1115x; 24.1115x over previous
import dataclasses
import functools

import jax
import jax.numpy as jnp
from jax import lax
from jax.experimental import pallas as pl
from jax.experimental.pallas import tpu as pltpu
from jax.experimental.pallas import tpu_sc as plsc
import numpy as np

B = 4096
V = 256
NB = 32
NKNOTS = NB + 1
MIN_BW = 1e-3
MIN_BH = 1e-3
MIN_D = 1e-3

NC = 2
NS = 16
L = 16
NW = NC * NS
N = B * V
PER_W = N // NW
CHUNK = 8192


def _tables_body(uw_ref, uh_ref, udp_ref, cumw_ref, cumh_ref, d_ref):
    tri = (lax.broadcasted_iota(jnp.int32, (NB, NKNOTS), 0)
           < lax.broadcasted_iota(jnp.int32, (NB, NKNOTS), 1)).astype(jnp.float32)
    col = lax.broadcasted_iota(jnp.int32, (V, NKNOTS), 1)

    def cum_table(u, min_size):
        p = jax.nn.softmax(u, axis=-1)
        p = min_size + (1.0 - min_size * NB) * p
        c = jnp.dot(p, tri, preferred_element_type=jnp.float32,
                    precision=lax.Precision.HIGHEST)
        return jnp.where(col == NB, 1.0, c)

    cumw_ref[...] = cum_table(uw_ref[...], MIN_BW)
    cumh_ref[...] = cum_table(uh_ref[...], MIN_BH)
    udp = udp_ref[...]
    d_ref[...] = MIN_D + jnp.log(1.0 + jnp.exp(udp))


def _compute_tables(uw, uh, udp):
    out_shape = jax.ShapeDtypeStruct((V, NKNOTS), jnp.float32)
    return pl.pallas_call(
        _tables_body,
        out_shape=(out_shape, out_shape, out_shape),
    )(uw, uh, udp)


_LN2 = 0.6931471805599453
_SQRT2 = 1.4142135623730951


def _log16(y):
    bits = plsc.bitcast(y, jnp.int32)
    e = lax.shift_right_logical(bits, 23) - 127
    m = plsc.bitcast((bits & 0x007FFFFF) | 0x3F800000, jnp.float32)
    big = m > _SQRT2
    m = jnp.where(big, m * 0.5, m)
    ef = e.astype(jnp.float32) + jnp.where(big, 1.0, 0.0)
    r = m - 1.0
    s = r / (2.0 + r)
    s2 = s * s
    p = 2.0 + s2 * (2.0 / 3.0 + s2 * (0.4 + s2 * (2.0 / 7.0)))
    return ef * _LN2 + s * p


def _sc_body(x_hbm, cumw_hbm, cumh_hbm, d_hbm, out_hbm, ld_hbm,
             cumw_v, cumh_v, d_v, x_v, out_v, ld_v):
    cid = lax.axis_index("c")
    sid = lax.axis_index("s")
    wid = sid * NC + cid
    base = wid * PER_W
    pltpu.sync_copy(cumw_hbm, cumw_v)
    pltpu.sync_copy(cumh_hbm, cumh_v)
    pltpu.sync_copy(d_hbm, d_v)

    @pl.loop(0, PER_W, step=CHUNK)
    def _chunk(c0):
        cbase = base + c0
        pltpu.sync_copy(x_hbm.at[pl.ds(cbase, CHUNK)], x_v)

        @pl.loop(0, CHUNK, step=L)
        def _vec(j):
            x_in = x_v[pl.ds(j, L)]
            vbase = (cbase + j) & (V - 1)
            tb = (vbase + lax.iota(jnp.int32, L)) * NKNOTS
            x = jnp.clip(x_in, 0.0, 1.0)

            idx = jnp.zeros((L,), jnp.int32)
            for step in (16, 8, 4, 2, 1):
                cand = idx + step
                bv = plsc.load_gather(cumw_v, [tb + cand])
                idx = jnp.where(x >= bv, cand, idx)

            g0 = tb + idx
            g1 = g0 + 1
            cw0 = plsc.load_gather(cumw_v, [g0])
            cw1 = plsc.load_gather(cumw_v, [g1])
            ch0 = plsc.load_gather(cumh_v, [g0])
            ch1 = plsc.load_gather(cumh_v, [g1])
            d0 = plsc.load_gather(d_v, [g0])
            d1 = plsc.load_gather(d_v, [g1])

            w = cw1 - cw0
            h = ch1 - ch0
            delta = h / w
            theta = (x - cw0) / w
            om = 1.0 - theta
            tom = theta * om
            th2 = theta * theta
            num = h * (delta * th2 + d0 * tom)
            den = delta + (d0 + d1 - 2.0 * delta) * tom
            out_in = ch0 + num / den
            dnum = delta * delta * (d1 * th2 + 2.0 * delta * tom + d0 * om * om)
            ld_in = _log16(dnum / (den * den))

            outside = (x_in < 0.0) | (x_in > 1.0)
            out_v[pl.ds(j, L)] = jnp.where(outside, x_in, out_in)
            ld_v[pl.ds(j, L)] = jnp.where(outside, 0.0, ld_in)

        pltpu.sync_copy(out_v, out_hbm.at[pl.ds(cbase, CHUNK)])
        pltpu.sync_copy(ld_v, ld_hbm.at[pl.ds(cbase, CHUNK)])


def _spline_sc(x_flat, cumw, cumh, d):
    mesh = plsc.VectorSubcoreMesh(core_axis_name="c", subcore_axis_name="s")
    flat = jax.ShapeDtypeStruct((N,), jnp.float32)
    cp = pltpu.CompilerParams()
    if "needs_layout_passes" in pltpu.CompilerParams.__dataclass_fields__:
        cp = dataclasses.replace(cp, needs_layout_passes=False)
    run = pl.kernel(
        _sc_body,
        out_type=(flat, flat),
        mesh=mesh,
        compiler_params=cp,
        scratch_types=[
            pltpu.VMEM((V * NKNOTS,), jnp.float32),
            pltpu.VMEM((V * NKNOTS,), jnp.float32),
            pltpu.VMEM((V * NKNOTS,), jnp.float32),
            pltpu.VMEM((CHUNK,), jnp.float32),
            pltpu.VMEM((CHUNK,), jnp.float32),
            pltpu.VMEM((CHUNK,), jnp.float32),
        ],
    )
    return run(x_flat, cumw, cumh, d)


@jax.jit
def kernel(inputs, unnormalized_widths, unnormalized_heights,
           unnormalized_derivatives):
    constant = float(np.log(np.exp(1.0 - MIN_D) - 1.0))
    udp = jnp.pad(unnormalized_derivatives, [(0, 0), (1, 1)],
                  constant_values=constant)
    cumw, cumh, d = _compute_tables(unnormalized_widths,
                                    unnormalized_heights, udp)
    out_flat, ld_flat = _spline_sc(inputs.reshape(N),
                                   cumw.reshape(V * NKNOTS),
                                   cumh.reshape(V * NKNOTS),
                                   d.reshape(V * NKNOTS))
    return out_flat.reshape(B, V), ld_flat.reshape(B, V)


# speedup vs baseline: 24.8319x; 1.0299x over previous
import dataclasses
import functools

import jax
import jax.numpy as jnp
from jax import lax
from jax.experimental import pallas as pl
from jax.experimental.pallas import tpu as pltpu
from jax.experimental.pallas import tpu_sc as plsc
import numpy as np

B = 4096
V = 256
NB = 32
NKNOTS = NB + 1
MIN_BW = 1e-3
MIN_BH = 1e-3
MIN_D = 1e-3

NC = 2
NS = 16
L = 16
NW = NC * NS
N = B * V
PER_W = N // NW
CHUNK = 8192


def _tables_body(uw_ref, uh_ref, udp_ref, cumw_ref, rw_ref, ch0_ref, h_ref,
                 d0_ref, d1_ref):
    tri = (lax.broadcasted_iota(jnp.int32, (NB, NKNOTS), 0)
           < lax.broadcasted_iota(jnp.int32, (NB, NKNOTS), 1)).astype(jnp.float32)
    col = lax.broadcasted_iota(jnp.int32, (V, NKNOTS), 1)

    def cum_table(u, min_size):
        p = jax.nn.softmax(u, axis=-1)
        p = min_size + (1.0 - min_size * NB) * p
        c = jnp.dot(p, tri, preferred_element_type=jnp.float32,
                    precision=lax.Precision.HIGHEST)
        return jnp.where(col == NB, 1.0, c)

    cumw = cum_table(uw_ref[...], MIN_BW)
    cumh = cum_table(uh_ref[...], MIN_BH)
    cumw_ref[...] = cumw
    rw_ref[...] = 1.0 / (cumw[:, 1:] - cumw[:, :NB])
    ch0_ref[...] = cumh[:, :NB]
    h_ref[...] = cumh[:, 1:] - cumh[:, :NB]
    udp = udp_ref[...]
    d = MIN_D + jnp.log(1.0 + jnp.exp(udp))
    d0_ref[...] = d[:, :NB]
    d1_ref[...] = d[:, 1:]


def _compute_tables(uw, uh, udp):
    knots = jax.ShapeDtypeStruct((V, NKNOTS), jnp.float32)
    bins = jax.ShapeDtypeStruct((V, NB), jnp.float32)
    return pl.pallas_call(
        _tables_body,
        out_shape=(knots, bins, bins, bins, bins, bins),
    )(uw, uh, udp)


_LN2 = 0.6931471805599453
_SQRT2 = 1.4142135623730951


def _log16(y):
    bits = plsc.bitcast(y, jnp.int32)
    e = lax.shift_right_logical(bits, 23) - 127
    m = plsc.bitcast((bits & 0x007FFFFF) | 0x3F800000, jnp.float32)
    big = m > _SQRT2
    m = jnp.where(big, m * 0.5, m)
    ef = e.astype(jnp.float32) + jnp.where(big, 1.0, 0.0)
    r = m - 1.0
    s = r / (2.0 + r)
    s2 = s * s
    p = 2.0 + s2 * (2.0 / 3.0 + s2 * (0.4 + s2 * (2.0 / 7.0)))
    return ef * _LN2 + s * p


UNROLL = 2


def _sc_body(x_hbm, cumw_hbm, rw_hbm, ch0_hbm, h_hbm, d0_hbm, d1_hbm,
             out_hbm, ld_hbm,
             cumw_v, rw_v, ch0_v, h_v, d0_v, d1_v, x_v, out_v, ld_v):
    cid = lax.axis_index("c")
    sid = lax.axis_index("s")
    wid = sid * NC + cid
    base = wid * PER_W
    pltpu.sync_copy(cumw_hbm, cumw_v)
    pltpu.sync_copy(rw_hbm, rw_v)
    pltpu.sync_copy(ch0_hbm, ch0_v)
    pltpu.sync_copy(h_hbm, h_v)
    pltpu.sync_copy(d0_hbm, d0_v)
    pltpu.sync_copy(d1_hbm, d1_v)

    def one_vec(j):
        x_in = x_v[pl.ds(j, L)]
        vbase = j & (V - 1)
        vvec = vbase + lax.iota(jnp.int32, L)
        tb = vvec * NKNOTS
        x = jnp.clip(x_in, 0.0, 1.0)

        idx = jnp.zeros((L,), jnp.int32)
        for step in (16, 8, 4, 2, 1):
            cand = idx + step
            bv = plsc.load_gather(cumw_v, [tb + cand])
            idx = jnp.where(x >= bv, cand, idx)

        g0 = tb + idx
        gb = vvec * NB + idx
        cw0 = plsc.load_gather(cumw_v, [g0])
        rw = plsc.load_gather(rw_v, [gb])
        ch0 = plsc.load_gather(ch0_v, [gb])
        h = plsc.load_gather(h_v, [gb])
        d0 = plsc.load_gather(d0_v, [gb])
        d1 = plsc.load_gather(d1_v, [gb])

        delta = h * rw
        theta = (x - cw0) * rw
        om = 1.0 - theta
        tom = theta * om
        th2 = theta * theta
        num = h * (delta * th2 + d0 * tom)
        den = delta + (d0 + d1 - 2.0 * delta) * tom
        out_in = ch0 + num / den
        dnum = delta * delta * (d1 * th2 + 2.0 * delta * tom + d0 * om * om)
        ld_in = _log16(dnum / (den * den))

        outside = (x_in < 0.0) | (x_in > 1.0)
        out_v[pl.ds(j, L)] = jnp.where(outside, x_in, out_in)
        ld_v[pl.ds(j, L)] = jnp.where(outside, 0.0, ld_in)

    @pl.loop(0, PER_W, step=CHUNK)
    def _chunk(c0):
        cbase = base + c0
        pltpu.sync_copy(x_hbm.at[pl.ds(cbase, CHUNK)], x_v)

        @pl.loop(0, CHUNK, step=UNROLL * L)
        def _vec(j):
            for u in range(UNROLL):
                one_vec(j + u * L)

        pltpu.sync_copy(out_v, out_hbm.at[pl.ds(cbase, CHUNK)])
        pltpu.sync_copy(ld_v, ld_hbm.at[pl.ds(cbase, CHUNK)])


def _spline_sc(x_flat, cumw, rw, ch0, h, d0, d1):
    mesh = plsc.VectorSubcoreMesh(core_axis_name="c", subcore_axis_name="s")
    flat = jax.ShapeDtypeStruct((N,), jnp.float32)
    cp = pltpu.CompilerParams()
    if "needs_layout_passes" in pltpu.CompilerParams.__dataclass_fields__:
        cp = dataclasses.replace(cp, needs_layout_passes=False)
    run = pl.kernel(
        _sc_body,
        out_type=(flat, flat),
        mesh=mesh,
        compiler_params=cp,
        scratch_types=[
            pltpu.VMEM((V * NKNOTS,), jnp.float32),
            pltpu.VMEM((V * NB,), jnp.float32),
            pltpu.VMEM((V * NB,), jnp.float32),
            pltpu.VMEM((V * NB,), jnp.float32),
            pltpu.VMEM((V * NB,), jnp.float32),
            pltpu.VMEM((V * NB,), jnp.float32),
            pltpu.VMEM((CHUNK,), jnp.float32),
            pltpu.VMEM((CHUNK,), jnp.float32),
            pltpu.VMEM((CHUNK,), jnp.float32),
        ],
    )
    return run(x_flat, cumw, rw, ch0, h, d0, d1)


@jax.jit
def kernel(inputs, unnormalized_widths, unnormalized_heights,
           unnormalized_derivatives):
    constant = float(np.log(np.exp(1.0 - MIN_D) - 1.0))
    udp = jnp.pad(unnormalized_derivatives, [(0, 0), (1, 1)],
                  constant_values=constant)
    cumw, rw, ch0, h, d0, d1 = _compute_tables(unnormalized_widths,
                                               unnormalized_heights, udp)
    out_flat, ld_flat = _spline_sc(inputs.reshape(N),
                                   cumw.reshape(V * NKNOTS),
                                   rw.reshape(V * NB),
                                   ch0.reshape(V * NB),
                                   h.reshape(V * NB),
                                   d0.reshape(V * NB),
                                   d1.reshape(V * NB))
    return out_flat.reshape(B, V), ld_flat.reshape(B, V)


# speedup vs baseline: 104.9818x; 4.2277x over previous
import dataclasses

import jax
import jax.numpy as jnp
from jax import lax
from jax.experimental import pallas as pl
from jax.experimental.pallas import tpu as pltpu
from jax.experimental.pallas import tpu_sc as plsc
import numpy as np

B = 4096
V = 256
NB = 32
NKNOTS = NB + 1
MIN_BW = 1e-3
MIN_BH = 1e-3
MIN_D = 1e-3

NC = 2
L = 16
NW = NC * 16
CHUNK = 8192


def _tables_body(uwt_ref, uht_ref, udpt_ref, cumw_ref, rw_ref, ch0_ref, h_ref,
                 d_ref):
    tri = (lax.broadcasted_iota(jnp.int32, (NKNOTS, NB), 1)
           < lax.broadcasted_iota(jnp.int32, (NKNOTS, NB), 0)).astype(jnp.float32)
    row = lax.broadcasted_iota(jnp.int32, (NKNOTS, V), 0)

    def cum_table(u, min_size):
        p = jax.nn.softmax(u, axis=0)
        p = min_size + (1.0 - min_size * NB) * p
        c = jnp.dot(tri, p, preferred_element_type=jnp.float32,
                    precision=lax.Precision.HIGHEST)
        return jnp.where(row == NB, 1.0, c)

    cumw = cum_table(uwt_ref[...], MIN_BW)
    cumh = cum_table(uht_ref[...], MIN_BH)
    cumw_ref[...] = cumw
    rw_ref[...] = 1.0 / (cumw[1:, :] - cumw[:NB, :])
    ch0_ref[...] = cumh[:NB, :]
    h_ref[...] = cumh[1:, :] - cumh[:NB, :]
    udp = udpt_ref[...]
    d_ref[...] = MIN_D + jnp.log(1.0 + jnp.exp(udp))


def _compute_tables(uwt, uht, udpt):
    knots = jax.ShapeDtypeStruct((NKNOTS, V), jnp.float32)
    bins = jax.ShapeDtypeStruct((NB, V), jnp.float32)
    return pl.pallas_call(
        _tables_body,
        out_shape=(knots, bins, bins, bins, knots),
    )(uwt, uht, udpt)


_LN2 = 0.6931471805599453
_SQRT2 = 1.4142135623730951


def _log16(y):
    bits = plsc.bitcast(y, jnp.int32)
    e = lax.shift_right_logical(bits, 23) - 127
    m = plsc.bitcast((bits & 0x007FFFFF) | 0x3F800000, jnp.float32)
    big = m > _SQRT2
    m = jnp.where(big, m * 0.5, m)
    ef = e.astype(jnp.float32) + jnp.where(big, 1.0, 0.0)
    r = m - 1.0
    s = r / (2.0 + r)
    s2 = s * s
    p = 2.0 + s2 * (2.0 / 3.0 + s2 * (0.4 + s2 * (2.0 / 7.0)))
    return ef * _LN2 + s * p


UNROLL = 4


R_SC = 2048
R_TC = B - R_SC
CHUNK_ROWS = CHUNK // V
ROWS_PER_TILE = R_SC // NW
NCHUNK = ROWS_PER_TILE // CHUNK_ROWS
TBLK = 512


def _sc_body(x_hbm, cumw_hbm, rw_hbm, ch0_hbm, h_hbm, d_hbm,
             out_hbm, ld_hbm,
             cumw_v, rw_v, ch0_v, h_v, d_v,
             x_b0, x_b1, out_b0, out_b1, ld_b0, ld_b1,
             semt, semx0, semx1, semo0, semo1, seml0, seml1):
    cid = lax.axis_index("c")
    sid = lax.axis_index("s")
    wid = sid * NC + cid
    base = wid * ROWS_PER_TILE
    x_b = (x_b0, x_b1)
    out_b = (out_b0, out_b1)
    ld_b = (ld_b0, ld_b1)
    semx = (semx0, semx1)
    semo = (semo0, semo1)
    seml = (seml0, seml1)

    def xsrc(i):
        return x_hbm.at[pl.ds(base + i * CHUNK_ROWS, CHUNK_ROWS), :]

    hx = [pltpu.async_copy(xsrc(0), x_b[0], semx[0]),
          pltpu.async_copy(xsrc(1), x_b[1], semx[1])]
    ht = [pltpu.async_copy(src, dst, semt)
          for src, dst in ((cumw_hbm, cumw_v), (rw_hbm, rw_v),
                           (ch0_hbm, ch0_v), (h_hbm, h_v), (d_hbm, d_v))]

    def one_vec(x_v, out_v, ld_v, r, c):
        x_in = x_v[r, pl.ds(c, L)]
        vvec = c + lax.iota(jnp.int32, L)
        x = jnp.clip(x_in, 0.0, 1.0)

        idx = jnp.zeros((L,), jnp.int32)
        for step in (16, 8, 4, 2, 1):
            cand = idx + step
            bv = plsc.load_gather(cumw_v, [cand, vvec])
            idx = jnp.where(x >= bv, cand, idx)

        cw0 = plsc.load_gather(cumw_v, [idx, vvec])
        rw = plsc.load_gather(rw_v, [idx, vvec])
        ch0 = plsc.load_gather(ch0_v, [idx, vvec])
        h = plsc.load_gather(h_v, [idx, vvec])
        d0 = plsc.load_gather(d_v, [idx, vvec])
        d1 = plsc.load_gather(d_v, [idx + 1, vvec])

        delta = h * rw
        theta = (x - cw0) * rw
        om = 1.0 - theta
        tom = theta * om
        th2 = theta * theta
        num = h * (delta * th2 + d0 * tom)
        den = delta + (d0 + d1 - 2.0 * delta) * tom
        out_in = ch0 + num / den
        dnum = delta * delta * (d1 * th2 + 2.0 * delta * tom + d0 * om * om)
        ld_in = _log16(dnum / (den * den))

        outside = (x_in < 0.0) | (x_in > 1.0)
        out_v[r, pl.ds(c, L)] = jnp.where(outside, x_in, out_in)
        ld_v[r, pl.ds(c, L)] = jnp.where(outside, 0.0, ld_in)

    for h_ in ht:
        h_.wait()

    scat = [None, None]
    for i in range(NCHUNK):
        b = i & 1
        hx[b].wait()
        if scat[b] is not None:
            scat[b][0].wait()
            scat[b][1].wait()

        @plsc.parallel_loop(0, CHUNK, step=L, unroll=UNROLL)
        def _vec(j, xv=x_b[b], ov=out_b[b], lv=ld_b[b]):
            one_vec(xv, ov, lv, lax.shift_right_logical(j, 8), j & (V - 1))

        if i + 2 < NCHUNK:
            hx[b] = pltpu.async_copy(xsrc(i + 2), x_b[b], semx[b])
        dst = pl.ds(base + i * CHUNK_ROWS, CHUNK_ROWS)
        scat[b] = (pltpu.async_copy(out_b[b], out_hbm.at[dst, :], semo[b]),
                   pltpu.async_copy(ld_b[b], ld_hbm.at[dst, :], seml[b]))
    for s in scat:
        s[0].wait()
        s[1].wait()


def _spline_sc(x, cumw, rw, ch0, h, d):
    mesh = plsc.VectorSubcoreMesh(core_axis_name="c", subcore_axis_name="s")
    cp = pltpu.CompilerParams()
    if "needs_layout_passes" in pltpu.CompilerParams.__dataclass_fields__:
        cp = dataclasses.replace(cp, needs_layout_passes=False)
    run = pl.kernel(
        _sc_body,
        out_type=(jax.ShapeDtypeStruct((R_SC, V), jnp.float32),
                  jax.ShapeDtypeStruct((R_SC, V), jnp.float32)),
        mesh=mesh,
        compiler_params=cp,
        scratch_types=[
            pltpu.VMEM((NKNOTS, V), jnp.float32),
            pltpu.VMEM((NB, V), jnp.float32),
            pltpu.VMEM((NB, V), jnp.float32),
            pltpu.VMEM((NB, V), jnp.float32),
            pltpu.VMEM((NKNOTS, V), jnp.float32),
            pltpu.VMEM((CHUNK_ROWS, V), jnp.float32),
            pltpu.VMEM((CHUNK_ROWS, V), jnp.float32),
            pltpu.VMEM((CHUNK_ROWS, V), jnp.float32),
            pltpu.VMEM((CHUNK_ROWS, V), jnp.float32),
            pltpu.VMEM((CHUNK_ROWS, V), jnp.float32),
            pltpu.VMEM((CHUNK_ROWS, V), jnp.float32),
            pltpu.SemaphoreType.DMA,
            pltpu.SemaphoreType.DMA,
            pltpu.SemaphoreType.DMA,
            pltpu.SemaphoreType.DMA,
            pltpu.SemaphoreType.DMA,
            pltpu.SemaphoreType.DMA,
            pltpu.SemaphoreType.DMA,
        ],
    )
    return run(x, cumw, rw, ch0, h, d)


def _tc_spline_body(x_ref, cumw_ref, rw_ref, ch0_ref, h_ref, d_ref,
                    out_ref, ld_ref):
    x_in = x_ref[...]
    x = jnp.clip(x_in, 0.0, 1.0)
    cumw = cumw_ref[...]
    rwt = rw_ref[...]
    ch0t = ch0_ref[...]
    ht = h_ref[...]
    dt = d_ref[...]

    cw0 = jnp.zeros_like(x)
    rw = jnp.broadcast_to(rwt[0], x.shape)
    ch0 = jnp.broadcast_to(ch0t[0], x.shape)
    h = jnp.broadcast_to(ht[0], x.shape)
    d0 = jnp.broadcast_to(dt[0], x.shape)
    d1 = jnp.broadcast_to(dt[1], x.shape)
    for k in range(1, NB):
        step = (x >= cumw[k]).astype(jnp.float32)
        cw0 = cw0 + step * (cumw[k] - cumw[k - 1])
        rw = rw + step * (rwt[k] - rwt[k - 1])
        ch0 = ch0 + step * (ch0t[k] - ch0t[k - 1])
        h = h + step * (ht[k] - ht[k - 1])
        d0 = d0 + step * (dt[k] - dt[k - 1])
        d1 = d1 + step * (dt[k + 1] - dt[k])

    delta = h * rw
    theta = (x - cw0) * rw
    om = 1.0 - theta
    tom = theta * om
    th2 = theta * theta
    num = h * (delta * th2 + d0 * tom)
    den = delta + (d0 + d1 - 2.0 * delta) * tom
    out_in = ch0 + num / den
    dnum = delta * delta * (d1 * th2 + 2.0 * delta * tom + d0 * om * om)
    ld_in = jnp.log(dnum) - 2.0 * jnp.log(den)

    outside = (x_in < 0.0) | (x_in > 1.0)
    out_ref[...] = jnp.where(outside, x_in, out_in)
    ld_ref[...] = jnp.where(outside, 0.0, ld_in)


def _spline_tc(x, cumw, rw, ch0, h, d):
    full = jax.ShapeDtypeStruct((B, V), jnp.float32)
    table_spec = pl.BlockSpec((NKNOTS, V), lambda i: (0, 0))
    bins_spec = pl.BlockSpec((NB, V), lambda i: (0, 0))
    tail_spec = pl.BlockSpec((TBLK, V), lambda i: (R_SC // TBLK + i, 0))
    return pl.pallas_call(
        _tc_spline_body,
        grid=(R_TC // TBLK,),
        in_specs=[
            tail_spec,
            table_spec, bins_spec, bins_spec, bins_spec, table_spec,
        ],
        out_specs=[tail_spec, tail_spec],
        out_shape=(full, full),
    )(x, cumw, rw, ch0, h, d)


@jax.jit
def kernel(inputs, unnormalized_widths, unnormalized_heights,
           unnormalized_derivatives):
    constant = float(np.log(np.exp(1.0 - MIN_D) - 1.0))
    udpt = jnp.pad(unnormalized_derivatives.T, [(1, 1), (0, 0)],
                   constant_values=constant)
    cumw, rw, ch0, h, d = _compute_tables(unnormalized_widths.T,
                                          unnormalized_heights.T, udpt)
    out_sc, ld_sc = _spline_sc(inputs, cumw, rw, ch0, h, d)
    out_tc, ld_tc = _spline_tc(inputs, cumw, rw, ch0, h, d)
    return (lax.dynamic_update_slice(out_tc, out_sc, (0, 0)),
            lax.dynamic_update_slice(ld_tc, ld_sc, (0, 0)))
